# Initial kernel scaffold; baseline (speedup 1.0000x reference)
import jax
import jax.numpy as jnp
from jax.experimental import pallas as pl


def kernel(xyz, emb, Wf1, bf1, Wf2, bf2, Wc1, bc1, Wc2, bc2, Wo1, bo1, Wo2, bo2, z):
    raise NotImplementedError("write your pallas kernel here")


# speedup vs baseline: 7.5994x; 7.5994x over previous
import functools

import jax
import jax.numpy as jnp
from jax import lax
from jax.experimental import pallas as pl
from jax.experimental.pallas import tpu as pltpu
from jax.experimental.pallas import tpu_sc as plsc

N_ATOMS = 4096
BOX = 40.0
CUT = 5.0
HIDDEN = 64
NG = 50
N_CONV = 3
NW = 32
APW = N_ATOMS // NW
CAP = 8192
EC = 128
TRASH = APW
ACC_ROWS = APW + 8
BS = 2048

_f32 = jnp.float32
_i32 = jnp.int32


def _ssp(x):
    return jnp.maximum(x, 0.0) + jnp.log1p(jnp.exp(-jnp.abs(x))) - 0.6931471805599453


def _full(v, dtype=_i32):
    return jnp.full((16,), v, dtype=dtype)


def _nbr_call(xs, ys, zs):
    mesh = plsc.VectorSubcoreMesh(core_axis_name="c", subcore_axis_name="s")

    @functools.partial(
        pl.kernel,
        mesh=mesh,
        compiler_params=pltpu.CompilerParams(needs_layout_passes=False),
        out_type=[
            jax.ShapeDtypeStruct((NW * CAP,), _i32),
            jax.ShapeDtypeStruct((NW * CAP,), _i32),
            jax.ShapeDtypeStruct((NW * CAP,), _f32),
            jax.ShapeDtypeStruct((NW * 16,), _i32),
        ],
        scratch_types=[
            pltpu.VMEM((N_ATOMS,), _f32),
            pltpu.VMEM((N_ATOMS,), _f32),
            pltpu.VMEM((N_ATOMS,), _f32),
            pltpu.VMEM((CAP,), _i32),
            pltpu.VMEM((CAP,), _i32),
            pltpu.VMEM((CAP,), _f32),
            pltpu.VMEM((16,), _i32),
        ],
    )
    def nbr(xs_hbm, ys_hbm, zs_hbm, src_hbm, dst_hbm, d2_hbm, cnt_hbm,
            x_v, y_v, z_v, src_v, dst_v, d2_v, cnt_v):
        wid = lax.axis_index("s") * 2 + lax.axis_index("c")
        pltpu.sync_copy(xs_hbm, x_v)
        pltpu.sync_copy(ys_hbm, y_v)
        pltpu.sync_copy(zs_hbm, z_v)

        def init_body(k, _):
            src_v[pl.ds(k * 16, 16)] = _full(0)
            dst_v[pl.ds(k * 16, 16)] = _full(TRASH)
            d2_v[pl.ds(k * 16, 16)] = _full(1.0e9, _f32)
            return 0

        lax.fori_loop(0, CAP // 16, init_body, 0, unroll=4)

        lane = lax.iota(_i32, 16)

        def row_body(a_loc, cnt):
            a = wid * APW + a_loc
            av = _full(a)
            xa = plsc.load_gather(x_v, [av])
            ya = plsc.load_gather(y_v, [av])
            za = plsc.load_gather(z_v, [av])

            def chunk_body(cq, cnt):
                j0 = cq * 16
                jvec = _full(j0) + lane
                dx = jnp.abs(x_v[pl.ds(j0, 16)] - xa)
                dy = jnp.abs(y_v[pl.ds(j0, 16)] - ya)
                dz = jnp.abs(z_v[pl.ds(j0, 16)] - za)
                dx = jnp.minimum(dx, BOX - dx)
                dy = jnp.minimum(dy, BOX - dy)
                dz = jnp.minimum(dz, BOX - dz)
                d2 = dx * dx + dy * dy + dz * dz
                msk = (d2 < CUT * CUT) & (jvec != av)
                plsc.store_compressed(src_v.at[pl.ds(cnt, 16)], jvec, mask=msk)
                plsc.store_compressed(dst_v.at[pl.ds(cnt, 16)], _full(0) + a_loc, mask=msk)
                plsc.store_compressed(d2_v.at[pl.ds(cnt, 16)], d2, mask=msk)
                pc = plsc.all_reduce_population_count(msk)
                cnt = jnp.minimum(cnt + pc[0], CAP - 16)
                return cnt

            return lax.fori_loop(0, N_ATOMS // 16, chunk_body, cnt, unroll=4)

        cnt = lax.fori_loop(0, APW, row_body, jnp.int32(0))

        pltpu.sync_copy(src_v, src_hbm.at[pl.ds(wid * CAP, CAP)])
        pltpu.sync_copy(dst_v, dst_hbm.at[pl.ds(wid * CAP, CAP)])
        pltpu.sync_copy(d2_v, d2_hbm.at[pl.ds(wid * CAP, CAP)])
        cnt_v[...] = jnp.full((16,), cnt, _i32)
        pltpu.sync_copy(cnt_v, cnt_hbm.at[pl.ds(wid * 16, 16)])

    return nbr(xs, ys, zs)


def _filter_kernel(first, z_ref, emb_ref, hprev_ref, msg_ref, Wc2_ref, bc2_ref,
                   d2_ref, Wf1T_ref, bf1_ref, Wf2T_ref, bf2_ref, Wc1_ref, bc1_ref,
                   f_ref, hj_ref, h_ref):
    @pl.when((pl.program_id(0) == 0) & (pl.program_id(1) == 0))
    def _():
        if first:
            zcol = z_ref[...]
            oh = (zcol == lax.broadcasted_iota(_i32, (N_ATOMS, 10), 1)).astype(_f32)
            h = lax.dot_general(oh, emb_ref[...], (((1,), (0,)), ((), ())),
                                preferred_element_type=_f32)
        else:
            m = lax.dot_general(msg_ref[...], Wc2_ref[...], (((1,), (0,)), ((), ())),
                                preferred_element_type=_f32)
            h = hprev_ref[...] + _ssp(m + bc2_ref[...])
        h_ref[...] = h
        hj = lax.dot_general(h, Wc1_ref[...], (((1,), (0,)), ((), ())),
                             preferred_element_type=_f32) + bc1_ref[...]
        hj_ref[...] = jnp.concatenate(
            [hj, jnp.zeros((N_ATOMS, 128 - HIDDEN), _f32)], axis=1)

    d2 = d2_ref[...].reshape(1, BS)
    d = jnp.sqrt(d2 + 1e-12)
    db = jnp.broadcast_to(d, (NG, BS))
    centers = lax.broadcasted_iota(_i32, (NG, BS), 0).astype(_f32) * (CUT / (NG - 1))
    delta = db - centers
    ef = jnp.exp(delta * delta * (-1.0 / (2.0 * (CUT / NG) ** 2)))
    u = _ssp(lax.dot_general(Wf1T_ref[...], ef, (((1,), (0,)), ((), ())),
                             preferred_element_type=_f32) + bf1_ref[...])
    fT = lax.dot_general(Wf2T_ref[...], u, (((1,), (0,)), ((), ())),
                         preferred_element_type=_f32) + bf2_ref[...]
    f_ref[...] = fT.reshape(1, HIDDEN, BS)


def _filter_call(first, z2, emb, hprev, msg, Wc2, bc2, d2, Wf1T, bf1c, Wf2T, bf2c,
                 Wc1, bc1r):
    grid = (NW, CAP // BS)
    const = lambda *_: (0, 0)
    in_specs = [
        pl.BlockSpec((N_ATOMS, 1), const),
        pl.BlockSpec((10, HIDDEN), const),
        pl.BlockSpec((N_ATOMS, HIDDEN), const),
        pl.BlockSpec((N_ATOMS, HIDDEN), const),
        pl.BlockSpec((HIDDEN, HIDDEN), const),
        pl.BlockSpec((1, HIDDEN), const),
        pl.BlockSpec((1, 1, BS), lambda w, b: (w * (CAP // BS) + b, 0, 0)),
        pl.BlockSpec((HIDDEN, NG), const),
        pl.BlockSpec((HIDDEN, 1), const),
        pl.BlockSpec((HIDDEN, HIDDEN), const),
        pl.BlockSpec((HIDDEN, 1), const),
        pl.BlockSpec((HIDDEN, HIDDEN), const),
        pl.BlockSpec((1, HIDDEN), const),
    ]
    out_specs = [
        pl.BlockSpec((1, HIDDEN, BS), lambda w, b: (w, 0, b)),
        pl.BlockSpec((N_ATOMS, 128), const),
        pl.BlockSpec((N_ATOMS, HIDDEN), const),
    ]
    out_shape = [
        jax.ShapeDtypeStruct((NW, HIDDEN, CAP), _f32),
        jax.ShapeDtypeStruct((N_ATOMS, 128), _f32),
        jax.ShapeDtypeStruct((N_ATOMS, HIDDEN), _f32),
    ]
    return pl.pallas_call(
        functools.partial(_filter_kernel, first),
        grid=grid, in_specs=in_specs, out_specs=out_specs, out_shape=out_shape,
    )(z2, emb, hprev, msg, Wc2, bc2, d2, Wf1T, bf1c, Wf2T, bf2c, Wc1, bc1r)


def _msg_call(src, dst, cnts, f, hj):
    mesh = plsc.VectorSubcoreMesh(core_axis_name="c", subcore_axis_name="s")

    @functools.partial(
        pl.kernel,
        mesh=mesh,
        compiler_params=pltpu.CompilerParams(needs_layout_passes=False),
        out_type=jax.ShapeDtypeStruct((N_ATOMS * HIDDEN,), _f32),
        scratch_types=[
            pltpu.VMEM((EC,), _i32),
            pltpu.VMEM((EC,), _i32),
            pltpu.VMEM((EC, 128), _f32),
            pltpu.VMEM((HIDDEN, EC), _f32),
            pltpu.VMEM((ACC_ROWS * HIDDEN,), _f32),
            pltpu.VMEM((16,), _i32),
            pltpu.SemaphoreType.DMA,
        ],
    )
    def msg_k(src_hbm, dst_hbm, cnt_hbm, f_hbm, hj_hbm, msg_hbm,
              idx_v, dst_v, rows_v, f_v, acc_v, cnt_v, sem):
        wid = lax.axis_index("s") * 2 + lax.axis_index("c")

        def zero_body(k, _):
            acc_v[pl.ds(k * 16, 16)] = jnp.zeros((16,), _f32)
            return 0

        lax.fori_loop(0, ACC_ROWS * HIDDEN // 16, zero_body, 0, unroll=4)

        pltpu.sync_copy(cnt_hbm.at[pl.ds(wid * 16, 16)], cnt_v)
        cnt = jnp.max(cnt_v[...])
        nchunks = (cnt + (EC - 1)) // EC
        lane = lax.iota(_i32, 16)

        def chunk_body(ch, _):
            e0 = ch * EC
            pltpu.sync_copy(src_hbm.at[pl.ds(wid * CAP + e0, EC)], idx_v)
            pltpu.sync_copy(dst_hbm.at[pl.ds(wid * CAP + e0, EC)], dst_v)
            pltpu.sync_copy(f_hbm.at[wid, :, pl.ds(e0, EC)], f_v)
            pltpu.async_copy(hj_hbm.at[idx_v], rows_v, sem).wait()

            def group_body(g, _):
                for k in range(16):
                    e = g * 16 + k
                    dloc = plsc.load_gather(dst_v, [_full(0) + e])
                    for q in range(HIDDEN // 16):
                        fq = plsc.load_gather(
                            f_v, [_full(q * 16) + lane, _full(0) + e])
                        hq = rows_v[e, pl.ds(q * 16, 16)]
                        plsc.addupdate_scatter(
                            acc_v, [dloc * HIDDEN + _full(q * 16) + lane], fq * hq)
                return 0

            lax.fori_loop(0, EC // 16, group_body, 0)
            return 0

        lax.fori_loop(0, nchunks, chunk_body, 0)
        pltpu.sync_copy(acc_v.at[pl.ds(0, APW * HIDDEN)],
                        msg_hbm.at[pl.ds(wid * APW * HIDDEN, APW * HIDDEN)])

    return msg_k(src, dst, cnts, f, hj)


def _final_kernel(h_ref, msg_ref, Wc2_ref, bc2_ref, Wo1_ref, bo1_ref,
                  Wo2_ref, bo2_ref, out_ref):
    m = lax.dot_general(msg_ref[...], Wc2_ref[...], (((1,), (0,)), ((), ())),
                        preferred_element_type=_f32)
    h = h_ref[...] + _ssp(m + bc2_ref[...])
    a1 = _ssp(lax.dot_general(h, Wo1_ref[...], (((1,), (0,)), ((), ())),
                              preferred_element_type=_f32) + bo1_ref[...])
    e = lax.dot_general(a1, Wo2_ref[...], (((1,), (0,)), ((), ())),
                        preferred_element_type=_f32) + bo2_ref[...]
    out_ref[...] = jnp.sum(e).reshape(1, 1)


def _final_call(h, msg, Wc2, bc2, Wo1, bo1r, Wo2, bo2r):
    return pl.pallas_call(
        _final_kernel,
        out_shape=jax.ShapeDtypeStruct((1, 1), _f32),
    )(h, msg, Wc2, bc2, Wo1, bo1r, Wo2, bo2r)


def kernel(xyz, emb, Wf1, bf1, Wf2, bf2, Wc1, bc1, Wc2, bc2, Wo1, bo1, Wo2, bo2, z):
    xyzf = xyz.astype(_f32)
    src, dst, d2, cnts = _nbr_call(xyzf[:, 0], xyzf[:, 1], xyzf[:, 2])

    z2 = z.astype(_i32).reshape(N_ATOMS, 1)
    dummy_h = jnp.zeros((N_ATOMS, HIDDEN), _f32)
    dummy_w = jnp.zeros((HIDDEN, HIDDEN), _f32)
    dummy_b = jnp.zeros((1, HIDDEN), _f32)

    h = dummy_h
    msg = dummy_h
    for l in range(N_CONV):
        first = l == 0
        f, hj, h = _filter_call(
            first, z2, emb,
            h, msg,
            dummy_w if first else Wc2[l - 1],
            dummy_b if first else bc2[l - 1].reshape(1, HIDDEN),
            d2.reshape(NW * (CAP // BS), 1, BS),
            Wf1[l].T, bf1[l].reshape(HIDDEN, 1),
            Wf2[l].T, bf2[l].reshape(HIDDEN, 1),
            Wc1[l], bc1[l].reshape(1, HIDDEN),
        )
        msg = _msg_call(src, dst, cnts, f, hj).reshape(N_ATOMS, HIDDEN)

    out = _final_call(h, msg, Wc2[N_CONV - 1], bc2[N_CONV - 1].reshape(1, HIDDEN),
                      Wo1, bo1.reshape(1, HIDDEN // 2), Wo2, bo2.reshape(1, 1))
    return out[0, 0]


# speedup vs baseline: 8.8400x; 1.1632x over previous
import functools

import jax
import jax.numpy as jnp
from jax import lax
from jax.experimental import pallas as pl
from jax.experimental.pallas import tpu as pltpu
from jax.experimental.pallas import tpu_sc as plsc

N_ATOMS = 4096
BOX = 40.0
CUT = 5.0
HIDDEN = 64
NG = 50
N_CONV = 3
NW = 32
APW = N_ATOMS // NW
CAP = 8192
EC = 128
TRASH = APW
ACC_ROWS = APW + 8
BS = 2048

_f32 = jnp.float32
_i32 = jnp.int32


def _ssp(x):
    return jnp.maximum(x, 0.0) + jnp.log1p(jnp.exp(-jnp.abs(x))) - 0.6931471805599453


def _full(v, dtype=_i32):
    return jnp.full((16,), v, dtype=dtype)


def _nbr_call(xs, ys, zs):
    mesh = plsc.VectorSubcoreMesh(core_axis_name="c", subcore_axis_name="s")

    @functools.partial(
        pl.kernel,
        mesh=mesh,
        compiler_params=pltpu.CompilerParams(needs_layout_passes=False),
        out_type=[
            jax.ShapeDtypeStruct((NW * CAP,), _i32),
            jax.ShapeDtypeStruct((NW * CAP,), _i32),
            jax.ShapeDtypeStruct((NW * CAP,), _f32),
            jax.ShapeDtypeStruct((NW * 16,), _i32),
        ],
        scratch_types=[
            pltpu.VMEM((N_ATOMS,), _f32),
            pltpu.VMEM((N_ATOMS,), _f32),
            pltpu.VMEM((N_ATOMS,), _f32),
            pltpu.VMEM((CAP,), _i32),
            pltpu.VMEM((CAP,), _i32),
            pltpu.VMEM((CAP,), _f32),
            pltpu.VMEM((16,), _i32),
        ],
    )
    def nbr(xs_hbm, ys_hbm, zs_hbm, src_hbm, dst_hbm, d2_hbm, cnt_hbm,
            x_v, y_v, z_v, src_v, dst_v, d2_v, cnt_v):
        wid = lax.axis_index("s") * 2 + lax.axis_index("c")
        pltpu.sync_copy(xs_hbm, x_v)
        pltpu.sync_copy(ys_hbm, y_v)
        pltpu.sync_copy(zs_hbm, z_v)

        def init_body(k, _):
            src_v[pl.ds(k * 16, 16)] = _full(0)
            dst_v[pl.ds(k * 16, 16)] = _full(TRASH)
            d2_v[pl.ds(k * 16, 16)] = _full(1.0e9, _f32)
            return 0

        lax.fori_loop(0, CAP // 16, init_body, 0, unroll=4)

        lane = lax.iota(_i32, 16)

        def row_body(a_loc, cnt):
            a = wid * APW + a_loc
            av = _full(a)
            xa = plsc.load_gather(x_v, [av])
            ya = plsc.load_gather(y_v, [av])
            za = plsc.load_gather(z_v, [av])

            def chunk_body(cq, cnt):
                j0 = cq * 16
                jvec = _full(j0) + lane
                dx = jnp.abs(x_v[pl.ds(j0, 16)] - xa)
                dy = jnp.abs(y_v[pl.ds(j0, 16)] - ya)
                dz = jnp.abs(z_v[pl.ds(j0, 16)] - za)
                dx = jnp.minimum(dx, BOX - dx)
                dy = jnp.minimum(dy, BOX - dy)
                dz = jnp.minimum(dz, BOX - dz)
                d2 = dx * dx + dy * dy + dz * dz
                msk = (d2 < CUT * CUT) & (jvec != av)
                plsc.store_compressed(src_v.at[pl.ds(cnt, 16)], jvec, mask=msk)
                plsc.store_compressed(dst_v.at[pl.ds(cnt, 16)], _full(0) + a_loc, mask=msk)
                plsc.store_compressed(d2_v.at[pl.ds(cnt, 16)], d2, mask=msk)
                pc = plsc.all_reduce_population_count(msk)
                cnt = jnp.minimum(cnt + pc[0], CAP - 16)
                return cnt

            return lax.fori_loop(0, N_ATOMS // 16, chunk_body, cnt, unroll=4)

        cnt = lax.fori_loop(0, APW, row_body, jnp.int32(0))

        pltpu.sync_copy(src_v, src_hbm.at[pl.ds(wid * CAP, CAP)])
        pltpu.sync_copy(dst_v, dst_hbm.at[pl.ds(wid * CAP, CAP)])
        pltpu.sync_copy(d2_v, d2_hbm.at[pl.ds(wid * CAP, CAP)])
        cnt_v[...] = jnp.full((16,), cnt, _i32)
        pltpu.sync_copy(cnt_v, cnt_hbm.at[pl.ds(wid * 16, 16)])

    return nbr(xs, ys, zs)


def _filter_kernel(first, z_ref, emb_ref, hprev_ref, msg_ref, Wc2_ref, bc2_ref,
                   d2_ref, Wf1T_ref, bf1_ref, Wf2T_ref, bf2_ref, Wc1_ref, bc1_ref,
                   f_ref, hj_ref, h_ref):
    @pl.when((pl.program_id(0) == 0) & (pl.program_id(1) == 0))
    def _():
        if first:
            zcol = z_ref[...]
            oh = (zcol == lax.broadcasted_iota(_i32, (N_ATOMS, 10), 1)).astype(_f32)
            h = lax.dot_general(oh, emb_ref[...], (((1,), (0,)), ((), ())),
                                preferred_element_type=_f32)
        else:
            m = lax.dot_general(msg_ref[...], Wc2_ref[...], (((1,), (0,)), ((), ())),
                                preferred_element_type=_f32)
            h = hprev_ref[...] + _ssp(m + bc2_ref[...])
        h_ref[...] = h
        hj = lax.dot_general(h, Wc1_ref[...], (((1,), (0,)), ((), ())),
                             preferred_element_type=_f32) + bc1_ref[...]
        hj_ref[...] = jnp.concatenate(
            [hj, jnp.zeros((N_ATOMS, 128 - HIDDEN), _f32)], axis=1)

    d2 = d2_ref[...].reshape(1, BS)
    d = jnp.sqrt(d2 + 1e-12)
    db = jnp.broadcast_to(d, (NG, BS))
    centers = lax.broadcasted_iota(_i32, (NG, BS), 0).astype(_f32) * (CUT / (NG - 1))
    delta = db - centers
    ef = jnp.exp(delta * delta * (-1.0 / (2.0 * (CUT / NG) ** 2)))
    u = _ssp(lax.dot_general(Wf1T_ref[...], ef, (((1,), (0,)), ((), ())),
                             preferred_element_type=_f32) + bf1_ref[...])
    fT = lax.dot_general(Wf2T_ref[...], u, (((1,), (0,)), ((), ())),
                         preferred_element_type=_f32) + bf2_ref[...]
    f_ref[...] = fT.reshape(1, HIDDEN, BS)


def _filter_call(first, z2, emb, hprev, msg, Wc2, bc2, d2, Wf1T, bf1c, Wf2T, bf2c,
                 Wc1, bc1r):
    grid = (NW, CAP // BS)
    const = lambda *_: (0, 0)
    in_specs = [
        pl.BlockSpec((N_ATOMS, 1), const),
        pl.BlockSpec((10, HIDDEN), const),
        pl.BlockSpec((N_ATOMS, HIDDEN), const),
        pl.BlockSpec((N_ATOMS, HIDDEN), const),
        pl.BlockSpec((HIDDEN, HIDDEN), const),
        pl.BlockSpec((1, HIDDEN), const),
        pl.BlockSpec((1, 1, BS), lambda w, b: (w * (CAP // BS) + b, 0, 0)),
        pl.BlockSpec((HIDDEN, NG), const),
        pl.BlockSpec((HIDDEN, 1), const),
        pl.BlockSpec((HIDDEN, HIDDEN), const),
        pl.BlockSpec((HIDDEN, 1), const),
        pl.BlockSpec((HIDDEN, HIDDEN), const),
        pl.BlockSpec((1, HIDDEN), const),
    ]
    out_specs = [
        pl.BlockSpec((1, HIDDEN, BS), lambda w, b: (w, 0, b)),
        pl.BlockSpec((N_ATOMS, 128), const),
        pl.BlockSpec((N_ATOMS, HIDDEN), const),
    ]
    out_shape = [
        jax.ShapeDtypeStruct((NW, HIDDEN, CAP), _f32),
        jax.ShapeDtypeStruct((N_ATOMS, 128), _f32),
        jax.ShapeDtypeStruct((N_ATOMS, HIDDEN), _f32),
    ]
    return pl.pallas_call(
        functools.partial(_filter_kernel, first),
        grid=grid, in_specs=in_specs, out_specs=out_specs, out_shape=out_shape,
    )(z2, emb, hprev, msg, Wc2, bc2, d2, Wf1T, bf1c, Wf2T, bf2c, Wc1, bc1r)


def _msg_call(src, dst, cnts, f, hj):
    mesh = plsc.VectorSubcoreMesh(core_axis_name="c", subcore_axis_name="s")

    @functools.partial(
        pl.kernel,
        mesh=mesh,
        compiler_params=pltpu.CompilerParams(needs_layout_passes=False),
        out_type=jax.ShapeDtypeStruct((N_ATOMS * HIDDEN,), _f32),
        scratch_types=[
            pltpu.VMEM((4 * EC,), _i32),
            pltpu.VMEM((4 * EC,), _i32),
            pltpu.VMEM((4 * HIDDEN, EC), _f32),
            pltpu.VMEM((2 * EC, 128), _f32),
            pltpu.VMEM((ACC_ROWS * HIDDEN,), _f32),
            pltpu.VMEM((16,), _i32),
            pltpu.SemaphoreType.DMA,
            pltpu.SemaphoreType.DMA,
            pltpu.SemaphoreType.DMA,
            pltpu.SemaphoreType.DMA,
            pltpu.SemaphoreType.DMA,
            pltpu.SemaphoreType.DMA,
        ],
    )
    def msg_k(src_hbm, dst_hbm, cnt_hbm, f_hbm, hj_hbm, msg_hbm,
              idx_v, dst_v, f_v, rows_v, acc_v, cnt_v,
              semL0, semL1, semL2, semL3, semG0, semG1):
        wid = lax.axis_index("s") * 2 + lax.axis_index("c")
        semL = [semL0, semL1, semL2, semL3]
        semG = [semG0, semG1]

        def zero_body(k, _):
            acc_v[pl.ds(k * 16, 16)] = jnp.zeros((16,), _f32)
            return 0

        lax.fori_loop(0, ACC_ROWS * HIDDEN // 16, zero_body, 0, unroll=4)

        pltpu.sync_copy(cnt_hbm.at[pl.ds(wid * 16, 16)], cnt_v)
        cnt = jnp.max(cnt_v[...])
        nchunks = (cnt + (EC - 1)) // EC
        lane = lax.iota(_i32, 16)

        def L_descr(c, sl):
            e0 = c * EC
            return (
                pltpu.make_async_copy(src_hbm.at[pl.ds(wid * CAP + e0, EC)],
                                      idx_v.at[pl.ds(sl * EC, EC)], semL[sl]),
                pltpu.make_async_copy(dst_hbm.at[pl.ds(wid * CAP + e0, EC)],
                                      dst_v.at[pl.ds(sl * EC, EC)], semL[sl]),
                pltpu.make_async_copy(f_hbm.at[wid, :, pl.ds(e0, EC)],
                                      f_v.at[pl.ds(sl * HIDDEN, HIDDEN), :], semL[sl]),
            )

        def startL(c, sl):
            for d in L_descr(c, sl):
                d.start()

        def waitL(c, sl):
            for d in L_descr(c, sl):
                d.wait()

        def G_descr(sl):
            return pltpu.make_async_copy(
                hj_hbm.at[idx_v.at[pl.ds(sl * EC, EC)]],
                rows_v.at[pl.ds((sl % 2) * EC, EC), :], semG[sl % 2])

        def compute(c, sl):
            slg = sl % 2
            rq = [_full(sl * HIDDEN + q * 16) + lane for q in range(4)]
            aq = [_full(q * 16) + lane for q in range(4)]

            def group_body(g, _):
                for k in range(16):
                    e = g * 16 + k
                    ev = _full(sl * EC) + e
                    dloc = plsc.load_gather(dst_v, [ev])
                    abase = dloc * HIDDEN
                    ecol = _full(0) + e
                    for q in range(4):
                        fq = plsc.load_gather(f_v, [rq[q], ecol])
                        hq = rows_v[slg * EC + e, pl.ds(q * 16, 16)]
                        plsc.addupdate_scatter(acc_v, [abase + aq[q]], fq * hq)
                return 0

            lax.fori_loop(0, EC // 16, group_body, 0)

        @pl.when(nchunks > 0)
        def _():
            startL(0, 0)

        @pl.when(nchunks > 1)
        def _():
            startL(1, 1)

        @pl.when(nchunks > 2)
        def _():
            startL(2, 2)

        @pl.when(nchunks > 0)
        def _():
            waitL(0, 0)
            G_descr(0).start()

        def super_body(qi, _):
            c0 = qi * 4
            for j in range(4):
                c = c0 + j

                @pl.when(c + 3 < nchunks)
                def _():
                    startL(c + 3, (j + 3) % 4)

                @pl.when(c + 1 < nchunks)
                def _():
                    waitL(c + 1, (j + 1) % 4)
                    G_descr((j + 1) % 4).start()

                @pl.when(c < nchunks)
                def _():
                    G_descr(j).wait()
                    compute(c, j)
            return 0

        lax.fori_loop(0, (nchunks + 3) // 4, super_body, 0)
        pltpu.sync_copy(acc_v.at[pl.ds(0, APW * HIDDEN)],
                        msg_hbm.at[pl.ds(wid * APW * HIDDEN, APW * HIDDEN)])

    return msg_k(src, dst, cnts, f, hj)


def _final_kernel(h_ref, msg_ref, Wc2_ref, bc2_ref, Wo1_ref, bo1_ref,
                  Wo2_ref, bo2_ref, out_ref):
    m = lax.dot_general(msg_ref[...], Wc2_ref[...], (((1,), (0,)), ((), ())),
                        preferred_element_type=_f32)
    h = h_ref[...] + _ssp(m + bc2_ref[...])
    a1 = _ssp(lax.dot_general(h, Wo1_ref[...], (((1,), (0,)), ((), ())),
                              preferred_element_type=_f32) + bo1_ref[...])
    e = lax.dot_general(a1, Wo2_ref[...], (((1,), (0,)), ((), ())),
                        preferred_element_type=_f32) + bo2_ref[...]
    out_ref[...] = jnp.sum(e).reshape(1, 1)


def _final_call(h, msg, Wc2, bc2, Wo1, bo1r, Wo2, bo2r):
    return pl.pallas_call(
        _final_kernel,
        out_shape=jax.ShapeDtypeStruct((1, 1), _f32),
    )(h, msg, Wc2, bc2, Wo1, bo1r, Wo2, bo2r)


def kernel(xyz, emb, Wf1, bf1, Wf2, bf2, Wc1, bc1, Wc2, bc2, Wo1, bo1, Wo2, bo2, z):
    xyzf = xyz.astype(_f32)
    src, dst, d2, cnts = _nbr_call(xyzf[:, 0], xyzf[:, 1], xyzf[:, 2])

    z2 = z.astype(_i32).reshape(N_ATOMS, 1)
    dummy_h = jnp.zeros((N_ATOMS, HIDDEN), _f32)
    dummy_w = jnp.zeros((HIDDEN, HIDDEN), _f32)
    dummy_b = jnp.zeros((1, HIDDEN), _f32)

    h = dummy_h
    msg = dummy_h
    for l in range(N_CONV):
        first = l == 0
        f, hj, h = _filter_call(
            first, z2, emb,
            h, msg,
            dummy_w if first else Wc2[l - 1],
            dummy_b if first else bc2[l - 1].reshape(1, HIDDEN),
            d2.reshape(NW * (CAP // BS), 1, BS),
            Wf1[l].T, bf1[l].reshape(HIDDEN, 1),
            Wf2[l].T, bf2[l].reshape(HIDDEN, 1),
            Wc1[l], bc1[l].reshape(1, HIDDEN),
        )
        msg = _msg_call(src, dst, cnts, f, hj).reshape(N_ATOMS, HIDDEN)

    out = _final_call(h, msg, Wc2[N_CONV - 1], bc2[N_CONV - 1].reshape(1, HIDDEN),
                      Wo1, bo1.reshape(1, HIDDEN // 2), Wo2, bo2.reshape(1, 1))
    return out[0, 0]


# speedup vs baseline: 9.6940x; 1.0966x over previous
import functools

import jax
import jax.numpy as jnp
from jax import lax
from jax.experimental import pallas as pl
from jax.experimental.pallas import tpu as pltpu
from jax.experimental.pallas import tpu_sc as plsc

N_ATOMS = 4096
BOX = 40.0
CUT = 5.0
HIDDEN = 64
NG = 50
N_CONV = 3
NW = 32
APW = N_ATOMS // NW
CAP = 8192
EC = 128
TRASH = APW
ACC_ROWS = APW + 8
BS = 2048

_f32 = jnp.float32
_i32 = jnp.int32


def _ssp(x):
    return jnp.maximum(x, 0.0) + jnp.log1p(jnp.exp(-jnp.abs(x))) - 0.6931471805599453


def _full(v, dtype=_i32):
    return jnp.full((16,), v, dtype=dtype)


def _nbr_call(xs, ys, zs):
    mesh = plsc.VectorSubcoreMesh(core_axis_name="c", subcore_axis_name="s")

    @functools.partial(
        pl.kernel,
        mesh=mesh,
        compiler_params=pltpu.CompilerParams(needs_layout_passes=False),
        out_type=[
            jax.ShapeDtypeStruct((NW * CAP,), _i32),
            jax.ShapeDtypeStruct((NW * CAP,), _i32),
            jax.ShapeDtypeStruct((NW * CAP,), _f32),
            jax.ShapeDtypeStruct((NW * 16,), _i32),
        ],
        scratch_types=[
            pltpu.VMEM((N_ATOMS,), _f32),
            pltpu.VMEM((N_ATOMS,), _f32),
            pltpu.VMEM((N_ATOMS,), _f32),
            pltpu.VMEM((CAP,), _i32),
            pltpu.VMEM((CAP,), _i32),
            pltpu.VMEM((CAP,), _f32),
            pltpu.VMEM((16,), _i32),
        ],
    )
    def nbr(xs_hbm, ys_hbm, zs_hbm, src_hbm, dst_hbm, d2_hbm, cnt_hbm,
            x_v, y_v, z_v, src_v, dst_v, d2_v, cnt_v):
        wid = lax.axis_index("s") * 2 + lax.axis_index("c")
        pltpu.sync_copy(xs_hbm, x_v)
        pltpu.sync_copy(ys_hbm, y_v)
        pltpu.sync_copy(zs_hbm, z_v)

        def init_body(k, _):
            src_v[pl.ds(k * 16, 16)] = _full(0)
            dst_v[pl.ds(k * 16, 16)] = _full(TRASH)
            d2_v[pl.ds(k * 16, 16)] = _full(1.0e9, _f32)
            return 0

        lax.fori_loop(0, CAP // 16, init_body, 0, unroll=4)

        lane = lax.iota(_i32, 16)

        def row_body(a_loc, cnt):
            a = wid * APW + a_loc
            av = _full(a)
            xa = plsc.load_gather(x_v, [av])
            ya = plsc.load_gather(y_v, [av])
            za = plsc.load_gather(z_v, [av])

            def chunk_body(cq, cnt):
                j0 = cq * 16
                jvec = _full(j0) + lane
                dx = jnp.abs(x_v[pl.ds(j0, 16)] - xa)
                dy = jnp.abs(y_v[pl.ds(j0, 16)] - ya)
                dz = jnp.abs(z_v[pl.ds(j0, 16)] - za)
                dx = jnp.minimum(dx, BOX - dx)
                dy = jnp.minimum(dy, BOX - dy)
                dz = jnp.minimum(dz, BOX - dz)
                d2 = dx * dx + dy * dy + dz * dz
                msk = (d2 < CUT * CUT) & (jvec != av)
                plsc.store_compressed(src_v.at[pl.ds(cnt, 16)], jvec, mask=msk)
                plsc.store_compressed(dst_v.at[pl.ds(cnt, 16)], _full(0) + a_loc, mask=msk)
                plsc.store_compressed(d2_v.at[pl.ds(cnt, 16)], d2, mask=msk)
                pc = plsc.all_reduce_population_count(msk)
                cnt = jnp.minimum(cnt + pc[0], CAP - 16)
                return cnt

            return lax.fori_loop(0, N_ATOMS // 16, chunk_body, cnt, unroll=4)

        cnt = lax.fori_loop(0, APW, row_body, jnp.int32(0))

        pltpu.sync_copy(src_v, src_hbm.at[pl.ds(wid * CAP, CAP)])
        pltpu.sync_copy(dst_v, dst_hbm.at[pl.ds(wid * CAP, CAP)])
        pltpu.sync_copy(d2_v, d2_hbm.at[pl.ds(wid * CAP, CAP)])
        cnt_v[...] = jnp.full((16,), cnt, _i32)
        pltpu.sync_copy(cnt_v, cnt_hbm.at[pl.ds(wid * 16, 16)])

    return nbr(xs, ys, zs)


def _filter_kernel(first, z_ref, emb_ref, hprev_ref, msg_ref, Wc2_ref, bc2_ref,
                   d2_ref, Wf1T_ref, bf1_ref, Wf2T_ref, bf2_ref, Wc1_ref, bc1_ref,
                   f_ref, hj_ref, h_ref):
    @pl.when((pl.program_id(0) == 0) & (pl.program_id(1) == 0))
    def _():
        if first:
            zcol = z_ref[...]
            oh = (zcol == lax.broadcasted_iota(_i32, (N_ATOMS, 10), 1)).astype(_f32)
            h = lax.dot_general(oh, emb_ref[...], (((1,), (0,)), ((), ())),
                                preferred_element_type=_f32)
        else:
            m = lax.dot_general(msg_ref[...], Wc2_ref[...], (((1,), (0,)), ((), ())),
                                preferred_element_type=_f32)
            h = hprev_ref[...] + _ssp(m + bc2_ref[...])
        h_ref[...] = h
        hj = lax.dot_general(h, Wc1_ref[...], (((1,), (0,)), ((), ())),
                             preferred_element_type=_f32) + bc1_ref[...]
        hj_ref[...] = jnp.concatenate(
            [hj, jnp.zeros((N_ATOMS, 128 - HIDDEN), _f32)], axis=1)

    d2 = d2_ref[...].reshape(1, BS)
    d = jnp.sqrt(d2 + 1e-12)
    db = jnp.broadcast_to(d, (NG, BS))
    centers = lax.broadcasted_iota(_i32, (NG, BS), 0).astype(_f32) * (CUT / (NG - 1))
    delta = db - centers
    ef = jnp.exp(delta * delta * (-1.0 / (2.0 * (CUT / NG) ** 2)))
    u = _ssp(lax.dot_general(Wf1T_ref[...], ef, (((1,), (0,)), ((), ())),
                             preferred_element_type=_f32) + bf1_ref[...])
    fT = lax.dot_general(Wf2T_ref[...], u, (((1,), (0,)), ((), ())),
                         preferred_element_type=_f32) + bf2_ref[...]
    f_ref[...] = fT.reshape(1, HIDDEN, BS)


def _filter_call(first, z2, emb, hprev, msg, Wc2, bc2, d2, Wf1T, bf1c, Wf2T, bf2c,
                 Wc1, bc1r):
    grid = (NW, CAP // BS)
    const = lambda *_: (0, 0)
    in_specs = [
        pl.BlockSpec((N_ATOMS, 1), const),
        pl.BlockSpec((10, HIDDEN), const),
        pl.BlockSpec((N_ATOMS, HIDDEN), const),
        pl.BlockSpec((N_ATOMS, HIDDEN), const),
        pl.BlockSpec((HIDDEN, HIDDEN), const),
        pl.BlockSpec((1, HIDDEN), const),
        pl.BlockSpec((1, 1, BS), lambda w, b: (w * (CAP // BS) + b, 0, 0)),
        pl.BlockSpec((HIDDEN, NG), const),
        pl.BlockSpec((HIDDEN, 1), const),
        pl.BlockSpec((HIDDEN, HIDDEN), const),
        pl.BlockSpec((HIDDEN, 1), const),
        pl.BlockSpec((HIDDEN, HIDDEN), const),
        pl.BlockSpec((1, HIDDEN), const),
    ]
    out_specs = [
        pl.BlockSpec((1, HIDDEN, BS), lambda w, b: (w, 0, b)),
        pl.BlockSpec((N_ATOMS, 128), const),
        pl.BlockSpec((N_ATOMS, HIDDEN), const),
    ]
    out_shape = [
        jax.ShapeDtypeStruct((NW, HIDDEN, CAP), _f32),
        jax.ShapeDtypeStruct((N_ATOMS, 128), _f32),
        jax.ShapeDtypeStruct((N_ATOMS, HIDDEN), _f32),
    ]
    return pl.pallas_call(
        functools.partial(_filter_kernel, first),
        grid=grid, in_specs=in_specs, out_specs=out_specs, out_shape=out_shape,
    )(z2, emb, hprev, msg, Wc2, bc2, d2, Wf1T, bf1c, Wf2T, bf2c, Wc1, bc1r)


def _msg_call(src, dst, cnts, f, hj):
    mesh = plsc.VectorSubcoreMesh(core_axis_name="c", subcore_axis_name="s")

    @functools.partial(
        pl.kernel,
        mesh=mesh,
        compiler_params=pltpu.CompilerParams(needs_layout_passes=False),
        out_type=jax.ShapeDtypeStruct((N_ATOMS * HIDDEN,), _f32),
        scratch_types=[
            pltpu.VMEM((4 * EC,), _i32),
            pltpu.VMEM((4 * EC,), _i32),
            pltpu.VMEM((4 * HIDDEN, EC), _f32),
            pltpu.VMEM((2 * EC, 128), _f32),
            pltpu.VMEM((ACC_ROWS * HIDDEN,), _f32),
            pltpu.VMEM((16,), _i32),
            pltpu.VMEM_SHARED((N_ATOMS, 128), _f32),
            pltpu.SemaphoreType.DMA,
            pltpu.SemaphoreType.DMA,
            pltpu.SemaphoreType.DMA,
            pltpu.SemaphoreType.DMA,
            pltpu.SemaphoreType.DMA,
            pltpu.SemaphoreType.DMA,
            pltpu.SemaphoreType.DMA,
        ],
    )
    def msg_k(src_hbm, dst_hbm, cnt_hbm, f_hbm, hj_hbm, msg_hbm,
              idx_v, dst_v, f_v, rows_v, acc_v, cnt_v, hjs,
              semL0, semL1, semL2, semL3, semG0, semG1, semS):
        wid = lax.axis_index("s") * 2 + lax.axis_index("c")
        sid = lax.axis_index("s")
        semL = [semL0, semL1, semL2, semL3]
        semG = [semG0, semG1]

        rows_per_tile = N_ATOMS // 16
        stage = pltpu.make_async_copy(
            hj_hbm.at[pl.ds(sid * rows_per_tile, rows_per_tile), :],
            hjs.at[pl.ds(sid * rows_per_tile, rows_per_tile), :], semS)
        stage.start()

        def zero_body(k, _):
            acc_v[pl.ds(k * 16, 16)] = jnp.zeros((16,), _f32)
            return 0

        lax.fori_loop(0, ACC_ROWS * HIDDEN // 16, zero_body, 0, unroll=4)
        stage.wait()
        plsc.subcore_barrier()

        pltpu.sync_copy(cnt_hbm.at[pl.ds(wid * 16, 16)], cnt_v)
        cnt = jnp.max(cnt_v[...])
        nchunks = (cnt + (EC - 1)) // EC
        lane = lax.iota(_i32, 16)

        def L_descr(c, sl):
            e0 = c * EC
            return (
                pltpu.make_async_copy(src_hbm.at[pl.ds(wid * CAP + e0, EC)],
                                      idx_v.at[pl.ds(sl * EC, EC)], semL[sl]),
                pltpu.make_async_copy(dst_hbm.at[pl.ds(wid * CAP + e0, EC)],
                                      dst_v.at[pl.ds(sl * EC, EC)], semL[sl]),
                pltpu.make_async_copy(f_hbm.at[wid, :, pl.ds(e0, EC)],
                                      f_v.at[pl.ds(sl * HIDDEN, HIDDEN), :], semL[sl]),
            )

        def startL(c, sl):
            for d in L_descr(c, sl):
                d.start()

        def waitL(c, sl):
            for d in L_descr(c, sl):
                d.wait()

        def G_descr(sl):
            return pltpu.make_async_copy(
                hjs.at[idx_v.at[pl.ds(sl * EC, EC)]],
                rows_v.at[pl.ds((sl % 2) * EC, EC), :], semG[sl % 2])

        def compute(c, sl):
            slg = sl % 2
            rq = [_full(sl * HIDDEN + q * 16) + lane for q in range(4)]
            aq = [_full(q * 16) + lane for q in range(4)]

            def group_body(g, _):
                for k in range(16):
                    e = g * 16 + k
                    ev = _full(sl * EC) + e
                    dloc = plsc.load_gather(dst_v, [ev])
                    abase = dloc * HIDDEN
                    ecol = _full(0) + e
                    for q in range(4):
                        fq = plsc.load_gather(f_v, [rq[q], ecol])
                        hq = rows_v[slg * EC + e, pl.ds(q * 16, 16)]
                        plsc.addupdate_scatter(acc_v, [abase + aq[q]], fq * hq)
                return 0

            lax.fori_loop(0, EC // 16, group_body, 0)

        @pl.when(nchunks > 0)
        def _():
            startL(0, 0)

        @pl.when(nchunks > 1)
        def _():
            startL(1, 1)

        @pl.when(nchunks > 2)
        def _():
            startL(2, 2)

        @pl.when(nchunks > 0)
        def _():
            waitL(0, 0)
            G_descr(0).start()

        def super_body(qi, _):
            c0 = qi * 4
            for j in range(4):
                c = c0 + j

                @pl.when(c + 3 < nchunks)
                def _():
                    startL(c + 3, (j + 3) % 4)

                @pl.when(c + 1 < nchunks)
                def _():
                    waitL(c + 1, (j + 1) % 4)
                    G_descr((j + 1) % 4).start()

                @pl.when(c < nchunks)
                def _():
                    G_descr(j).wait()
                    compute(c, j)
            return 0

        lax.fori_loop(0, (nchunks + 3) // 4, super_body, 0)
        pltpu.sync_copy(acc_v.at[pl.ds(0, APW * HIDDEN)],
                        msg_hbm.at[pl.ds(wid * APW * HIDDEN, APW * HIDDEN)])

    return msg_k(src, dst, cnts, f, hj)


def _final_kernel(h_ref, msg_ref, Wc2_ref, bc2_ref, Wo1_ref, bo1_ref,
                  Wo2_ref, bo2_ref, out_ref):
    m = lax.dot_general(msg_ref[...], Wc2_ref[...], (((1,), (0,)), ((), ())),
                        preferred_element_type=_f32)
    h = h_ref[...] + _ssp(m + bc2_ref[...])
    a1 = _ssp(lax.dot_general(h, Wo1_ref[...], (((1,), (0,)), ((), ())),
                              preferred_element_type=_f32) + bo1_ref[...])
    e = lax.dot_general(a1, Wo2_ref[...], (((1,), (0,)), ((), ())),
                        preferred_element_type=_f32) + bo2_ref[...]
    out_ref[...] = jnp.sum(e).reshape(1, 1)


def _final_call(h, msg, Wc2, bc2, Wo1, bo1r, Wo2, bo2r):
    return pl.pallas_call(
        _final_kernel,
        out_shape=jax.ShapeDtypeStruct((1, 1), _f32),
    )(h, msg, Wc2, bc2, Wo1, bo1r, Wo2, bo2r)


def kernel(xyz, emb, Wf1, bf1, Wf2, bf2, Wc1, bc1, Wc2, bc2, Wo1, bo1, Wo2, bo2, z):
    xyzf = xyz.astype(_f32)
    src, dst, d2, cnts = _nbr_call(xyzf[:, 0], xyzf[:, 1], xyzf[:, 2])

    z2 = z.astype(_i32).reshape(N_ATOMS, 1)
    dummy_h = jnp.zeros((N_ATOMS, HIDDEN), _f32)
    dummy_w = jnp.zeros((HIDDEN, HIDDEN), _f32)
    dummy_b = jnp.zeros((1, HIDDEN), _f32)

    h = dummy_h
    msg = dummy_h
    for l in range(N_CONV):
        first = l == 0
        f, hj, h = _filter_call(
            first, z2, emb,
            h, msg,
            dummy_w if first else Wc2[l - 1],
            dummy_b if first else bc2[l - 1].reshape(1, HIDDEN),
            d2.reshape(NW * (CAP // BS), 1, BS),
            Wf1[l].T, bf1[l].reshape(HIDDEN, 1),
            Wf2[l].T, bf2[l].reshape(HIDDEN, 1),
            Wc1[l], bc1[l].reshape(1, HIDDEN),
        )
        msg = _msg_call(src, dst, cnts, f, hj).reshape(N_ATOMS, HIDDEN)

    out = _final_call(h, msg, Wc2[N_CONV - 1], bc2[N_CONV - 1].reshape(1, HIDDEN),
                      Wo1, bo1.reshape(1, HIDDEN // 2), Wo2, bo2.reshape(1, 1))
    return out[0, 0]
